# Initial kernel scaffold; baseline (speedup 1.0000x reference)
import jax
import jax.numpy as jnp
from jax.experimental import pallas as pl


def kernel(x, Wq, Wk, Wv, Wo, Wk1, Wv1, Wk2, Wv2):
    raise NotImplementedError("write your pallas kernel here")


# speedup vs baseline: 122.0334x; 122.0334x over previous
import functools

import jax
import jax.numpy as jnp
from jax import lax
from jax.experimental import pallas as pl

B, S, DM, H = 1, 2048, 768, 12
D = DM // H
G = 32
N1 = S // G
TOP1 = 8
WIN = 512
TQ = 256
NT = S // TQ
BPT = TQ // G
NEG = -1e30


def _mm(a, b, dims):
    return lax.dot_general(a.astype(jnp.bfloat16), b.astype(jnp.bfloat16),
                           dims, preferred_element_type=jnp.float32)


def _qkv_body(x_ref, wq_ref, wk_ref, wv_ref, cos_ref, sin_ref,
              q_ref, k_ref, v_ref):
    xt = x_ref[...]
    cos = cos_ref[...]
    sin = sin_ref[...]
    qf = _mm(xt, wq_ref[...], (((1,), (1,)), ((), ())))
    kf = _mm(xt, wk_ref[...], (((1,), (1,)), ((), ())))
    vf = _mm(xt, wv_ref[...], (((1,), (1,)), ((), ())))
    for h in range(H):
        qh = qf[:, h * D:(h + 1) * D]
        kh = kf[:, h * D:(h + 1) * D]
        qrot = jnp.concatenate([-qh[:, D // 2:], qh[:, :D // 2]], axis=1)
        krot = jnp.concatenate([-kh[:, D // 2:], kh[:, :D // 2]], axis=1)
        q_ref[h] = qh * cos + qrot * sin
        k_ref[h] = kh * cos + krot * sin
        v_ref[h] = vf[:, h * D:(h + 1) * D]


def _route_body(q_ref, kr_ref, wk1_ref, sel_ref):
    k1 = _mm(kr_ref[...], wk1_ref[...], (((1,), (1,)), ((), ())))
    s = lax.dot_general(q_ref[...], k1, (((1,), (1,)), ((), ())),
                        preferred_element_type=jnp.float32, precision=lax.Precision.HIGHEST)
    iota = lax.broadcasted_iota(jnp.int32, (S, N1), 1)
    selected = jnp.zeros((S, N1), dtype=jnp.bool_)
    for _ in range(TOP1):
        m = jnp.max(s, axis=1, keepdims=True)
        ism = s >= m
        first_idx = jnp.min(jnp.where(ism, iota, N1), axis=1, keepdims=True)
        first = iota == first_idx
        selected = selected | first
        s = jnp.where(first, NEG, s)
    sel_ref[...] = selected.astype(jnp.float32)


def _attn_body(q_ref, k_ref, v_ref, sel_ref, o_ref):
    i = pl.program_id(1)
    scale = D ** -0.5
    qt = q_ref[...] * scale
    selb = sel_ref[...]
    rowg = lax.broadcasted_iota(jnp.int32, (TQ, TQ), 0) + i * TQ
    colg0 = lax.broadcasted_iota(jnp.int32, (TQ, TQ), 1)
    eb = lax.broadcasted_iota(jnp.int32, (N1, TQ), 0)
    ec = lax.broadcasted_iota(jnp.int32, (N1, TQ), 1) // G

    def step(j, carry):
        m, l, acc = carry
        kt = k_ref[pl.ds(j * TQ, TQ), :]
        vt = v_ref[pl.ds(j * TQ, TQ), :]
        s = _mm(qt, kt, (((1,), (1,)), ((), ())))
        ej = (eb == j * BPT + ec).astype(jnp.float32)
        sel_exp = _mm(selb, ej, (((1,), (0,)), ((), ())))
        colg = colg0 + j * TQ
        allowed = (colg <= rowg) & ((colg >= rowg - (WIN - 1)) |
                                    (sel_exp > 0.5))
        s = jnp.where(allowed, s, NEG)
        m2 = jnp.maximum(m, jnp.max(s, axis=1, keepdims=True))
        alpha = jnp.exp(m - m2)
        p = jnp.exp(s - m2)
        l2 = l * alpha + jnp.sum(p, axis=1, keepdims=True)
        acc2 = acc * alpha + _mm(p, vt, (((1,), (0,)), ((), ())))
        return m2, l2, acc2

    m0 = jnp.full((TQ, 1), NEG, dtype=jnp.float32)
    l0 = jnp.zeros((TQ, 1), dtype=jnp.float32)
    a0 = jnp.zeros((TQ, D), dtype=jnp.float32)
    m, l, acc = lax.fori_loop(0, i + 1, step, (m0, l0, a0))
    o_ref[...] = acc / l


def _proj_body(o_ref, wo_ref, y_ref):
    acc = jnp.zeros((TQ, DM), dtype=jnp.float32)
    for h in range(H):
        acc = acc + _mm(o_ref[h], wo_ref[:, h * D:(h + 1) * D],
                        (((1,), (1,)), ((), ())))
    y_ref[...] = acc


def _pipeline(x2, wq, wk, wv, wo, wk1, interpret=False):
    inv_freq = 1.0 / (10000.0 ** (jnp.arange(0, D, 2, dtype=jnp.float32) / D))
    t = jnp.arange(S, dtype=jnp.float32)
    freqs = jnp.outer(t, inv_freq)
    emb = jnp.concatenate([freqs, freqs], axis=-1)
    cos = jnp.cos(emb)
    sin = jnp.sin(emb)

    q, k, v = pl.pallas_call(
        _qkv_body,
        grid=(NT,),
        in_specs=[
            pl.BlockSpec((TQ, DM), lambda i: (i, 0)),
            pl.BlockSpec((DM, DM), lambda i: (0, 0)),
            pl.BlockSpec((DM, DM), lambda i: (0, 0)),
            pl.BlockSpec((DM, DM), lambda i: (0, 0)),
            pl.BlockSpec((TQ, D), lambda i: (i, 0)),
            pl.BlockSpec((TQ, D), lambda i: (i, 0)),
        ],
        out_specs=[
            pl.BlockSpec((H, TQ, D), lambda i: (0, i, 0)),
            pl.BlockSpec((H, TQ, D), lambda i: (0, i, 0)),
            pl.BlockSpec((H, TQ, D), lambda i: (0, i, 0)),
        ],
        out_shape=[jax.ShapeDtypeStruct((H, S, D), jnp.float32)] * 3,
        interpret=interpret,
    )(x2, wq, wk, wv, cos, sin)

    kr = k.reshape(H, N1, G * D)

    sel = pl.pallas_call(
        _route_body,
        grid=(H,),
        in_specs=[
            pl.BlockSpec((None, S, D), lambda h: (h, 0, 0)),
            pl.BlockSpec((None, N1, G * D), lambda h: (h, 0, 0)),
            pl.BlockSpec((D, G * D), lambda h: (0, 0)),
        ],
        out_specs=pl.BlockSpec((None, S, N1), lambda h: (h, 0, 0)),
        out_shape=jax.ShapeDtypeStruct((H, S, N1), jnp.float32),
        interpret=interpret,
    )(q, kr, wk1)

    o = pl.pallas_call(
        _attn_body,
        grid=(H, NT),
        in_specs=[
            pl.BlockSpec((None, TQ, D), lambda h, i: (h, i, 0)),
            pl.BlockSpec((None, S, D), lambda h, i: (h, 0, 0)),
            pl.BlockSpec((None, S, D), lambda h, i: (h, 0, 0)),
            pl.BlockSpec((None, TQ, N1), lambda h, i: (h, i, 0)),
        ],
        out_specs=pl.BlockSpec((None, TQ, D), lambda h, i: (h, i, 0)),
        out_shape=jax.ShapeDtypeStruct((H, S, D), jnp.float32),
        interpret=interpret,
    )(q, k, v, sel)

    y = pl.pallas_call(
        _proj_body,
        grid=(NT,),
        in_specs=[
            pl.BlockSpec((H, TQ, D), lambda i: (0, i, 0)),
            pl.BlockSpec((DM, DM), lambda i: (0, 0)),
        ],
        out_specs=pl.BlockSpec((TQ, DM), lambda i: (i, 0)),
        out_shape=jax.ShapeDtypeStruct((S, DM), jnp.float32),
        interpret=interpret,
    )(o, wo)
    return y


@jax.jit
def kernel(x, Wq, Wk, Wv, Wo, Wk1, Wv1, Wk2, Wv2):
    del Wv1, Wk2, Wv2
    y = _pipeline(x[0], Wq, Wk, Wv, Wo, Wk1)
    return y[None]


# speedup vs baseline: 232.6692x; 1.9066x over previous
import jax
import jax.numpy as jnp
from jax import lax
from jax.experimental import pallas as pl

B, S, DM, H = 1, 2048, 768, 12
D = DM // H
G = 32
N1 = S // G
TOP1 = 8
WIN = 512
TQ = 256
NT = S // TQ
TA = 512
NA = S // TA
BPT = TA // G
NEG = -1e30


def _mm(a, b, dims):
    return lax.dot_general(a.astype(jnp.bfloat16), b.astype(jnp.bfloat16),
                           dims, preferred_element_type=jnp.float32)


def _qkv_body(x_ref, wq_ref, wk_ref, wv_ref, cos_ref, sin_ref,
              q_ref, k_ref, v_ref):
    xt = x_ref[...]
    cos = cos_ref[...]
    sin = sin_ref[...]
    qf = _mm(xt, wq_ref[...], (((1,), (1,)), ((), ())))
    kf = _mm(xt, wk_ref[...], (((1,), (1,)), ((), ())))
    vf = _mm(xt, wv_ref[...], (((1,), (1,)), ((), ())))
    for h in range(H):
        qh = qf[:, h * D:(h + 1) * D]
        kh = kf[:, h * D:(h + 1) * D]
        qrot = jnp.concatenate([-qh[:, D // 2:], qh[:, :D // 2]], axis=1)
        krot = jnp.concatenate([-kh[:, D // 2:], kh[:, :D // 2]], axis=1)
        q_ref[h] = qh * cos + qrot * sin
        k_ref[h] = kh * cos + krot * sin
        v_ref[h] = vf[:, h * D:(h + 1) * D]


def _route_body(q_ref, kr_ref, wk1_ref, sel_ref):
    k1 = _mm(kr_ref[...], wk1_ref[...], (((1,), (1,)), ((), ())))
    s = lax.dot_general(k1, q_ref[...], (((1,), (1,)), ((), ())),
                        preferred_element_type=jnp.float32,
                        precision=lax.Precision.HIGHEST)
    iota = lax.broadcasted_iota(jnp.int32, (N1, S), 0)
    selected = jnp.zeros((N1, S), dtype=jnp.bool_)
    for _ in range(TOP1):
        m = jnp.max(s, axis=0, keepdims=True)
        ism = s >= m
        first_idx = jnp.min(jnp.where(ism, iota, N1), axis=0, keepdims=True)
        first = iota == first_idx
        selected = selected | first
        s = jnp.where(first, NEG, s)
    sel_ref[...] = selected.astype(jnp.float32)


def _attn_body(q_ref, k_ref, v_ref, sel_ref, o_ref):
    i = pl.program_id(1)
    scale = D ** -0.5
    qt = q_ref[...] * scale
    selt = sel_ref[...]
    ri = lax.broadcasted_iota(jnp.int32, (TA, TA), 0)
    ci = lax.broadcasted_iota(jnp.int32, (TA, TA), 1)
    causal_pen = jnp.where(ci <= ri, 0.0, NEG).astype(jnp.float32)
    anti_pen = jnp.where(ci > ri, 0.0, NEG).astype(jnp.float32)
    eb = lax.broadcasted_iota(jnp.int32, (N1, TA), 0)
    ec = lax.broadcasted_iota(jnp.int32, (N1, TA), 1) // G

    def tile(j, carry, mode):
        m, l, acc = carry
        kt = k_ref[pl.ds(j * TA, TA), :]
        vt = v_ref[pl.ds(j * TA, TA), :]
        s = _mm(qt, kt, (((1,), (1,)), ((), ())))
        if mode != "diag":
            ej = (eb == j * BPT + ec).astype(jnp.float32)
            sel_exp = _mm(selt, ej, (((0,), (0,)), ((), ())))
            sel_pen = (sel_exp - 1.0) * -NEG
            if mode == "win":
                s = s + jnp.maximum(anti_pen, sel_pen)
            else:
                s = s + sel_pen
        else:
            s = s + causal_pen
        m2 = jnp.maximum(m, jnp.max(s, axis=1, keepdims=True))
        alpha = jnp.exp(m - m2)
        p = jnp.exp(s - m2)
        l2 = l * alpha + jnp.sum(p, axis=1, keepdims=True)
        acc2 = acc * alpha + _mm(p, vt, (((1,), (0,)), ((), ())))
        return m2, l2, acc2

    m0 = jnp.full((TA, 1), NEG, dtype=jnp.float32)
    l0 = jnp.zeros((TA, 1), dtype=jnp.float32)
    a0 = jnp.zeros((TA, D), dtype=jnp.float32)
    carry = (m0, l0, a0)
    carry = lax.fori_loop(0, jnp.maximum(i - 1, 0),
                          lambda j, c: tile(j, c, "far"), carry)
    carry = lax.cond(i >= 1,
                     lambda c: tile(i - 1, c, "win"),
                     lambda c: c, carry)
    m, l, acc = tile(i, carry, "diag")
    o_ref[...] = acc / l


def _proj_body(o_ref, wo_ref, y_ref):
    acc = jnp.zeros((TQ, DM), dtype=jnp.float32)
    for h in range(H):
        acc = acc + _mm(o_ref[h], wo_ref[:, h * D:(h + 1) * D],
                        (((1,), (1,)), ((), ())))
    y_ref[...] = acc


def _pipeline(x2, wq, wk, wv, wo, wk1, interpret=False):
    inv_freq = 1.0 / (10000.0 ** (jnp.arange(0, D, 2, dtype=jnp.float32) / D))
    t = jnp.arange(S, dtype=jnp.float32)
    freqs = jnp.outer(t, inv_freq)
    emb = jnp.concatenate([freqs, freqs], axis=-1)
    cos = jnp.cos(emb)
    sin = jnp.sin(emb)

    q, k, v = pl.pallas_call(
        _qkv_body,
        grid=(NT,),
        in_specs=[
            pl.BlockSpec((TQ, DM), lambda i: (i, 0)),
            pl.BlockSpec((DM, DM), lambda i: (0, 0)),
            pl.BlockSpec((DM, DM), lambda i: (0, 0)),
            pl.BlockSpec((DM, DM), lambda i: (0, 0)),
            pl.BlockSpec((TQ, D), lambda i: (i, 0)),
            pl.BlockSpec((TQ, D), lambda i: (i, 0)),
        ],
        out_specs=[
            pl.BlockSpec((H, TQ, D), lambda i: (0, i, 0)),
            pl.BlockSpec((H, TQ, D), lambda i: (0, i, 0)),
            pl.BlockSpec((H, TQ, D), lambda i: (0, i, 0)),
        ],
        out_shape=[jax.ShapeDtypeStruct((H, S, D), jnp.float32)] * 3,
        interpret=interpret,
    )(x2, wq, wk, wv, cos, sin)

    kr = k.reshape(H, N1, G * D)

    sel = pl.pallas_call(
        _route_body,
        grid=(H,),
        in_specs=[
            pl.BlockSpec((None, S, D), lambda h: (h, 0, 0)),
            pl.BlockSpec((None, N1, G * D), lambda h: (h, 0, 0)),
            pl.BlockSpec((D, G * D), lambda h: (0, 0)),
        ],
        out_specs=pl.BlockSpec((None, N1, S), lambda h: (h, 0, 0)),
        out_shape=jax.ShapeDtypeStruct((H, N1, S), jnp.float32),
        interpret=interpret,
    )(q, kr, wk1)

    o = pl.pallas_call(
        _attn_body,
        grid=(H, NA),
        in_specs=[
            pl.BlockSpec((None, TA, D), lambda h, i: (h, i, 0)),
            pl.BlockSpec((None, S, D), lambda h, i: (h, 0, 0)),
            pl.BlockSpec((None, S, D), lambda h, i: (h, 0, 0)),
            pl.BlockSpec((None, N1, TA), lambda h, i: (h, 0, i)),
        ],
        out_specs=pl.BlockSpec((None, TA, D), lambda h, i: (h, i, 0)),
        out_shape=jax.ShapeDtypeStruct((H, S, D), jnp.float32),
        interpret=interpret,
    )(q, k, v, sel)

    y = pl.pallas_call(
        _proj_body,
        grid=(NT,),
        in_specs=[
            pl.BlockSpec((H, TQ, D), lambda i: (0, i, 0)),
            pl.BlockSpec((DM, DM), lambda i: (0, 0)),
        ],
        out_specs=pl.BlockSpec((TQ, DM), lambda i: (i, 0)),
        out_shape=jax.ShapeDtypeStruct((S, DM), jnp.float32),
        interpret=interpret,
    )(o, wo)
    return y


@jax.jit
def kernel(x, Wq, Wk, Wv, Wo, Wk1, Wv1, Wk2, Wv2):
    del Wv1, Wk2, Wv2
    y = _pipeline(x[0], Wq, Wk, Wv, Wo, Wk1)
    return y[None]
